# baseline (speedup 1.0000x reference)
import functools

import jax
import jax.numpy as jnp
from jax import lax
from jax.experimental import pallas as pl
from jax.experimental.pallas import tpu as pltpu
from jax.experimental.pallas import tpu_sc as plsc

B = 16384
LF = 20
V = 1000
D = 133
NP = 67
NC, NS = 2, 16
NW = NC * NS
BW = B // NW
CB = 128


@functools.partial(
    pl.kernel,
    mesh=plsc.VectorSubcoreMesh(core_axis_name="c", subcore_axis_name="s"),
    out_type=jax.ShapeDtypeStruct((D, B), jnp.float32),
    compiler_params=pltpu.CompilerParams(use_tc_tiling_on_sc=True,
                                         needs_layout_passes=False),
    scratch_types=[
        pltpu.VMEM((NP * V,), jnp.int32),
        pltpu.VMEM((LF, CB), jnp.int32),
        pltpu.VMEM((D, CB), jnp.float32),
    ],
)
def _fg_pool(idx_hbm, tab_hbm, out_hbm, tab_v, idx_v, obuf):
    wid = lax.axis_index("s") * NC + lax.axis_index("c")
    pltpu.sync_copy(tab_hbm, tab_v)

    def chunk(i, _):
        b0 = wid * BW + i * CB
        pltpu.sync_copy(idx_hbm.at[:, pl.ds(b0, CB)], idx_v)

        def group(qb, _):
            bb = qb * 16
            idxs = [idx_v[g, pl.ds(bb, 16)] for g in range(LF)]

            def accum(cp):
                word = tab_v.at[pl.ds(cp * V, V)]
                vals = [plsc.bitcast(plsc.load_gather(word, [idxs[g]]),
                                     jnp.bfloat16) for g in range(LF)]
                while len(vals) > 1:
                    vals = [vals[k] + vals[k + 1]
                            for k in range(0, len(vals) - 1, 2)] + (
                        [vals[-1]] if len(vals) % 2 else [])
                return plsc.unpack(vals[0],
                                   format=plsc.PackFormat.INTERLEAVED)

            for cp in range(NP - 1):
                lo, hi = accum(cp)
                obuf[2 * cp, pl.ds(bb, 16)] = lo
                obuf[2 * cp + 1, pl.ds(bb, 16)] = hi
            lo, _ = accum(NP - 1)
            obuf[D - 1, pl.ds(bb, 16)] = lo
            return 0

        lax.fori_loop(0, CB // 16, group, 0)
        pltpu.sync_copy(obuf, out_hbm.at[:, pl.ds(b0, CB)])
        return 0

    lax.fori_loop(0, BW // CB, chunk, 0)


def kernel(fg_indices, fg_embedding):
    idx_t = fg_indices.astype(jnp.int32).T
    tab_t = jnp.pad(fg_embedding.T * jnp.float32(1.0 / LF),
                    ((0, 1), (0, 0)))
    folded = tab_t.astype(jnp.bfloat16).reshape(NP, 2, V).transpose(0, 2, 1)
    packed = lax.bitcast_convert_type(folded, jnp.int32)
    out_t = _fg_pool(idx_t, packed.reshape(NP * V))
    return out_t.T


# speedup vs baseline: 1.6335x; 1.6335x over previous
import functools

import jax
import jax.numpy as jnp
from jax import lax
from jax.experimental import pallas as pl
from jax.experimental.pallas import tpu as pltpu
from jax.experimental.pallas import tpu_sc as plsc

B = 16384
LF = 20
V = 1000
D = 133
NP = 67
NC, NS = 2, 16
NW = NC * NS
BW = B // NW
CB = 128


@functools.partial(
    pl.kernel,
    mesh=plsc.VectorSubcoreMesh(core_axis_name="c", subcore_axis_name="s"),
    out_type=jax.ShapeDtypeStruct((D, B), jnp.float32),
    compiler_params=pltpu.CompilerParams(use_tc_tiling_on_sc=True,
                                         needs_layout_passes=False),
    scratch_types=[
        pltpu.VMEM((NP * V,), jnp.int32),
        pltpu.VMEM((LF, CB), jnp.int32),
        pltpu.VMEM((D, CB), jnp.float32),
    ],
)
def _fg_pool(idx_hbm, tab_hbm, out_hbm, tab_v, idx_v, obuf):
    wid = lax.axis_index("s") * NC + lax.axis_index("c")
    pltpu.sync_copy(tab_hbm, tab_v)

    def chunk(i, _):
        b0 = wid * BW + i * CB
        pltpu.sync_copy(idx_hbm.at[:, pl.ds(b0, CB)], idx_v)

        def group(qb, _):
            bb = qb * 16
            idxs = [idx_v[g, pl.ds(bb, 16)] for g in range(LF)]

            def accum(cp):
                word = tab_v.at[pl.ds(cp * V, V)]
                vals = [plsc.bitcast(plsc.load_gather(word, [idxs[g]]),
                                     jnp.bfloat16) for g in range(LF)]
                while len(vals) > 1:
                    vals = [vals[k] + vals[k + 1]
                            for k in range(0, len(vals) - 1, 2)] + (
                        [vals[-1]] if len(vals) % 2 else [])
                return plsc.unpack(vals[0],
                                   format=plsc.PackFormat.INTERLEAVED)

            def pair_body(u, _):
                for k in range(6):
                    cp = 6 * u + k
                    lo, hi = accum(cp)
                    obuf[2 * cp, pl.ds(bb, 16)] = lo
                    obuf[2 * cp + 1, pl.ds(bb, 16)] = hi
                return 0

            lax.fori_loop(0, (NP - 1) // 6, pair_body, 0)
            lo, _ = accum(NP - 1)
            obuf[D - 1, pl.ds(bb, 16)] = lo
            return 0

        lax.fori_loop(0, CB // 16, group, 0)
        pltpu.sync_copy(obuf, out_hbm.at[:, pl.ds(b0, CB)])
        return 0

    lax.fori_loop(0, BW // CB, chunk, 0)


def kernel(fg_indices, fg_embedding):
    idx_t = fg_indices.astype(jnp.int32).T
    tab_t = jnp.pad(fg_embedding.T * jnp.float32(1.0 / LF),
                    ((0, 1), (0, 0)))
    folded = tab_t.astype(jnp.bfloat16).reshape(NP, 2, V).transpose(0, 2, 1)
    packed = lax.bitcast_convert_type(folded, jnp.int32)
    out_t = _fg_pool(idx_t, packed.reshape(NP * V))
    return out_t.T


# speedup vs baseline: 1.6355x; 1.0012x over previous
import functools

import jax
import jax.numpy as jnp
from jax import lax
from jax.experimental import pallas as pl
from jax.experimental.pallas import tpu as pltpu
from jax.experimental.pallas import tpu_sc as plsc

B = 16384
LF = 20
V = 1000
D = 133
NP = 67
NC, NS = 2, 16
NW = NC * NS
BW = B // NW
CB = 128


@functools.partial(
    pl.kernel,
    mesh=plsc.VectorSubcoreMesh(core_axis_name="c", subcore_axis_name="s"),
    out_type=jax.ShapeDtypeStruct((D, B), jnp.float32),
    compiler_params=pltpu.CompilerParams(use_tc_tiling_on_sc=True,
                                         needs_layout_passes=False),
    scratch_types=[
        pltpu.VMEM((NP * V,), jnp.int32),
        pltpu.VMEM((LF, CB), jnp.int32),
        pltpu.VMEM((D, CB), jnp.float32),
    ],
)
def _fg_pool(idx_hbm, tab_hbm, out_hbm, tab_v, idx_v, obuf):
    wid = lax.axis_index("s") * NC + lax.axis_index("c")
    pltpu.sync_copy(tab_hbm, tab_v)

    def chunk(i, _):
        b0 = wid * BW + i * CB
        pltpu.sync_copy(idx_hbm.at[:, pl.ds(b0, CB)], idx_v)

        def group(qb, _):
            bb = qb * 16
            idxs = [idx_v[g, pl.ds(bb, 16)] for g in range(LF)]

            def accum(cp):
                word = tab_v.at[pl.ds(cp * V, V)]
                vals = [plsc.bitcast(plsc.load_gather(word, [idxs[g]]),
                                     jnp.bfloat16) for g in range(LF)]
                while len(vals) > 1:
                    vals = [vals[k] + vals[k + 1]
                            for k in range(0, len(vals) - 1, 2)] + (
                        [vals[-1]] if len(vals) % 2 else [])
                return plsc.unpack(vals[0],
                                   format=plsc.PackFormat.INTERLEAVED)

            def pair_body(u, _):
                for k in range(3):
                    cp = 3 * u + k
                    lo, hi = accum(cp)
                    obuf[2 * cp, pl.ds(bb, 16)] = lo
                    obuf[2 * cp + 1, pl.ds(bb, 16)] = hi
                return 0

            lax.fori_loop(0, (NP - 1) // 3, pair_body, 0)
            lo, _ = accum(NP - 1)
            obuf[D - 1, pl.ds(bb, 16)] = lo
            return 0

        lax.fori_loop(0, CB // 16, group, 0)
        pltpu.sync_copy(obuf, out_hbm.at[:, pl.ds(b0, CB)])
        return 0

    lax.fori_loop(0, BW // CB, chunk, 0)


def kernel(fg_indices, fg_embedding):
    idx_t = fg_indices.astype(jnp.int32).T
    tab_t = jnp.pad(fg_embedding.T * jnp.float32(1.0 / LF),
                    ((0, 1), (0, 0)))
    folded = tab_t.astype(jnp.bfloat16).reshape(NP, 2, V).transpose(0, 2, 1)
    packed = lax.bitcast_convert_type(folded, jnp.int32)
    out_t = _fg_pool(idx_t, packed.reshape(NP * V))
    return out_t.T


# speedup vs baseline: 1.6672x; 1.0194x over previous
import functools

import jax
import jax.numpy as jnp
from jax import lax
from jax.experimental import pallas as pl
from jax.experimental.pallas import tpu as pltpu
from jax.experimental.pallas import tpu_sc as plsc

B = 16384
LF = 20
V = 1000
D = 133
NP = 67
NC, NS = 2, 16
NW = NC * NS
BW = B // NW
CB = 256


@functools.partial(
    pl.kernel,
    mesh=plsc.VectorSubcoreMesh(core_axis_name="c", subcore_axis_name="s"),
    out_type=jax.ShapeDtypeStruct((D, B), jnp.float32),
    compiler_params=pltpu.CompilerParams(use_tc_tiling_on_sc=True,
                                         needs_layout_passes=False),
    scratch_types=[
        pltpu.VMEM((NP * V,), jnp.int32),
        pltpu.VMEM((LF, CB), jnp.int32),
        pltpu.VMEM((D, CB), jnp.float32),
    ],
)
def _fg_pool(idx_hbm, tab_hbm, out_hbm, tab_v, idx_v, obuf):
    wid = lax.axis_index("s") * NC + lax.axis_index("c")
    pltpu.sync_copy(tab_hbm, tab_v)

    def chunk(i, _):
        b0 = wid * BW + i * CB
        pltpu.sync_copy(idx_hbm.at[:, pl.ds(b0, CB)], idx_v)

        def group(qb, _):
            bb = qb * 16
            idxs = [idx_v[g, pl.ds(bb, 16)] for g in range(LF)]

            def accum(cp):
                word = tab_v.at[pl.ds(cp * V, V)]
                vals = [plsc.bitcast(plsc.load_gather(word, [idxs[g]]),
                                     jnp.bfloat16) for g in range(LF)]
                while len(vals) > 1:
                    vals = [vals[k] + vals[k + 1]
                            for k in range(0, len(vals) - 1, 2)] + (
                        [vals[-1]] if len(vals) % 2 else [])
                return plsc.unpack(vals[0],
                                   format=plsc.PackFormat.INTERLEAVED)

            def pair_body(u, _):
                for k in range(3):
                    cp = 3 * u + k
                    lo, hi = accum(cp)
                    obuf[2 * cp, pl.ds(bb, 16)] = lo
                    obuf[2 * cp + 1, pl.ds(bb, 16)] = hi
                return 0

            lax.fori_loop(0, (NP - 1) // 3, pair_body, 0)
            lo, _ = accum(NP - 1)
            obuf[D - 1, pl.ds(bb, 16)] = lo
            return 0

        lax.fori_loop(0, CB // 16, group, 0)
        pltpu.sync_copy(obuf, out_hbm.at[:, pl.ds(b0, CB)])
        return 0

    lax.fori_loop(0, BW // CB, chunk, 0)


def kernel(fg_indices, fg_embedding):
    idx_t = fg_indices.astype(jnp.int32).T
    tab_t = jnp.pad(fg_embedding.T * jnp.float32(1.0 / LF),
                    ((0, 1), (0, 0)))
    folded = tab_t.astype(jnp.bfloat16).reshape(NP, 2, V).transpose(0, 2, 1)
    packed = lax.bitcast_convert_type(folded, jnp.int32)
    out_t = _fg_pool(idx_t, packed.reshape(NP * V))
    return out_t.T
